# baseline (speedup 1.0000x reference)
import functools

import jax
import jax.numpy as jnp
from jax import lax
from jax.experimental import pallas as pl
from jax.experimental.pallas import tpu as pltpu
from jax.experimental.pallas import tpu_sc as plsc

_C = 256
_HS = (200, 100, 50, 25)
_SCALES = (0.25, 0.125, 0.0625, 0.03125)
_OUT = 7
_SR = 2
_P = _OUT * _SR
_OFFSETS = (0, 40000, 50000, 52500)
_S = 53125
_S_PAD = 53128
_NB = _OUT * _OUT
_K = 16
_RPB = _NB * _K
_BC = 7
_GR = _BC * _K
_NCHUNK = _NB // _BC
_LANES = 16
_NW = 32
_ROI_F = _NB * _C


def _build_table(feats):
    rows = [f[0].transpose(1, 2, 0).reshape(h * h, _C)
            for f, h in zip(feats, _HS)]
    rows.append(jnp.zeros((_S_PAD - _S, _C), jnp.float32))
    return jnp.concatenate(rows, axis=0)


def _indices_weights(rois, level):
    n = rois.shape[0]
    lvl = level.astype(jnp.int32)
    scales = jnp.array(_SCALES, jnp.float32)
    sizes = jnp.array(_HS, jnp.float32)
    offs = jnp.array(_OFFSETS, jnp.int32)
    wl_i = jnp.array(_HS, jnp.int32)
    sc = scales[lvl]
    hl = sizes[lvl]
    x1 = rois[:, 1] * sc
    y1 = rois[:, 2] * sc
    x2 = rois[:, 3] * sc
    y2 = rois[:, 4] * sc
    roi_w = jnp.maximum(x2 - x1, 1.0)
    roi_h = jnp.maximum(y2 - y1, 1.0)
    bin_h = roi_h / _OUT
    bin_w = roi_w / _OUT
    off = (jnp.arange(_P, dtype=jnp.float32) + 0.5) / _SR
    ys = jnp.clip(y1[:, None] + off[None, :] * bin_h[:, None], 0.0,
                  hl[:, None] - 1.0)
    xs = jnp.clip(x1[:, None] + off[None, :] * bin_w[:, None], 0.0,
                  hl[:, None] - 1.0)
    y0f = jnp.floor(ys)
    x0f = jnp.floor(xs)
    ly = ys - y0f
    lx = xs - x0f
    y0 = y0f.astype(jnp.int32)
    x0 = x0f.astype(jnp.int32)
    hi = (hl[:, None] - 1.0).astype(jnp.int32)
    y1i = jnp.minimum(y0 + 1, hi)
    x1i = jnp.minimum(x0 + 1, hi)
    base = offs[lvl][:, None]
    w = wl_i[lvl][:, None]
    rb = jnp.stack([base + y0 * w, base + y1i * w], axis=1)
    xx = jnp.stack([x0, x1i], axis=1)
    wy = jnp.stack([1.0 - ly, ly], axis=1)
    wx = jnp.stack([1.0 - lx, lx], axis=1)

    def expand(a, b, op):
        t = op(a[:, :, :, None, None], b[:, None, None, :, :])
        t = jnp.transpose(t, (0, 2, 4, 1, 3))
        t = t.reshape(n, _OUT, _SR, _OUT, _SR, 2, 2)
        t = jnp.transpose(t, (0, 1, 3, 2, 4, 5, 6))
        return t.reshape(n, _NB, _K)

    idx = expand(rb, xx, lambda u, v: u + v)
    wgt = expand(wy, wx, lambda u, v: u * v) * (1.0 / (_SR * _SR))
    wgt = jnp.broadcast_to(wgt[..., None], (n, _NB, _K, _LANES))
    return idx, wgt.reshape(n, _NB, _K * _LANES)


def _sc_kernel(idx_hbm, w_hbm, table_hbm, out_hbm,
               idx_v, w_v, rows_v, out_v, sem):
    wid = lax.axis_index("s") * 2 + lax.axis_index("c")
    rpw = idx_hbm.shape[0] // (_NW * _RPB)

    def roi_body(t, _):
        roi = wid * rpw + t
        base = roi * _RPB
        pltpu.sync_copy(idx_hbm.at[pl.ds(base, _RPB)], idx_v)
        pltpu.sync_copy(w_hbm.at[pl.ds(base * _LANES, _RPB * _LANES)], w_v)

        def chunk_body(cidx, _):
            pltpu.async_copy(
                table_hbm.at[idx_v.at[pl.ds(cidx * _GR, _GR)]],
                rows_v, sem).wait()

            def bin_body(bb, _):
                row0 = bb * _K
                accs = [jnp.zeros((_LANES,), jnp.float32)
                        for _ in range(_C // _LANES)]
                for r in range(_K):
                    wv = w_v[pl.ds((cidx * _GR + bb * _K + r) * _LANES,
                                   _LANES)]
                    for ch in range(_C // _LANES):
                        accs[ch] = accs[ch] + wv * rows_v[
                            row0 + r, pl.ds(ch * _LANES, _LANES)]
                ob = (cidx * _BC + bb) * _C
                for ch in range(_C // _LANES):
                    out_v[pl.ds(ob + ch * _LANES, _LANES)] = accs[ch]
                return 0

            lax.fori_loop(0, _BC, bin_body, 0)
            return 0

        lax.fori_loop(0, _NCHUNK, chunk_body, 0)
        pltpu.sync_copy(out_v, out_hbm.at[pl.ds(roi * _ROI_F, _ROI_F)])
        return 0

    lax.fori_loop(0, rpw, roi_body, 0)


@jax.jit
def _roi_align(input_0, input_1, input_2, input_3, rois, level):
    n = rois.shape[0]
    n_pad = ((n + _NW - 1) // _NW) * _NW
    table = _build_table((input_0, input_1, input_2, input_3))
    idx, wgt = _indices_weights(rois, level)
    idx = jnp.pad(idx, ((0, n_pad - n), (0, 0), (0, 0))).reshape(-1)
    wgt = jnp.pad(wgt, ((0, n_pad - n), (0, 0), (0, 0))).reshape(-1)

    mesh = plsc.VectorSubcoreMesh(core_axis_name="c", subcore_axis_name="s")
    run = pl.kernel(
        _sc_kernel,
        out_type=jax.ShapeDtypeStruct((n_pad * _ROI_F,), jnp.float32),
        mesh=mesh,
        scratch_types=[
            pltpu.VMEM((_RPB,), jnp.int32),
            pltpu.VMEM((_RPB * _LANES,), jnp.float32),
            pltpu.VMEM((_GR, _C), jnp.float32),
            pltpu.VMEM((_ROI_F,), jnp.float32),
            pltpu.SemaphoreType.DMA,
        ],
    )
    out = run(idx, wgt, table)
    out = out.reshape(n_pad, _OUT, _OUT, _C)[:n]
    return jnp.transpose(out, (0, 3, 1, 2))


def kernel(input_0, input_1, input_2, input_3, rois, rois_counts, level):
    del rois_counts
    return _roi_align(input_0, input_1, input_2, input_3, rois, level)


# speedup vs baseline: 2.7538x; 2.7538x over previous
import jax
import jax.numpy as jnp
from jax import lax
from jax.experimental import pallas as pl
from jax.experimental.pallas import tpu as pltpu
from jax.experimental.pallas import tpu_sc as plsc

_C = 256
_HS = (200, 100, 50, 25)
_SCALES = (0.25, 0.125, 0.0625, 0.03125)
_OUT = 7
_SR = 2
_P = _OUT * _SR
_OFFSETS = (0, 40000, 50000, 52500)
_S = 53125
_S_PAD = 53128
_NB = _OUT * _OUT
_K = 16
_RPB = _NB * _K
_BC = 7
_GR = _BC * _K
_NCHUNK = _NB // _BC
_LANES = 16
_NW = 32
_ROI_F = _NB * _C
_BB = 4
_FROWS = _BB * _BB
_R_GIDX = 16
_R_LIDX = 32
_REC = _R_LIDX + _RPB


def _build_table(feats):
    rows = [f[0].transpose(1, 2, 0).reshape(h * h, _C)
            for f, h in zip(feats, _HS)]
    rows.append(jnp.zeros((_S_PAD - _S, _C), jnp.float32))
    return jnp.concatenate(rows, axis=0)


def _records(rois, level):
    n = rois.shape[0]
    lvl = level.astype(jnp.int32)
    scales = jnp.array(_SCALES, jnp.float32)
    sizes = jnp.array(_HS, jnp.float32)
    offs = jnp.array(_OFFSETS, jnp.int32)
    wl_i = jnp.array(_HS, jnp.int32)
    sc = scales[lvl]
    hl = sizes[lvl]
    x1 = rois[:, 1] * sc
    y1 = rois[:, 2] * sc
    x2 = rois[:, 3] * sc
    y2 = rois[:, 4] * sc
    roi_w = jnp.maximum(x2 - x1, 1.0)
    roi_h = jnp.maximum(y2 - y1, 1.0)
    bin_h = roi_h / _OUT
    bin_w = roi_w / _OUT
    off = (jnp.arange(_P, dtype=jnp.float32) + 0.5) / _SR
    ys = jnp.clip(y1[:, None] + off[None, :] * bin_h[:, None], 0.0,
                  hl[:, None] - 1.0)
    xs = jnp.clip(x1[:, None] + off[None, :] * bin_w[:, None], 0.0,
                  hl[:, None] - 1.0)
    y0f = jnp.floor(ys)
    x0f = jnp.floor(xs)
    ly = ys - y0f
    lx = xs - x0f
    y0 = y0f.astype(jnp.int32)
    x0 = x0f.astype(jnp.int32)
    hi = (hl[:, None] - 1.0).astype(jnp.int32)
    y1i = jnp.minimum(y0 + 1, hi)
    x1i = jnp.minimum(x0 + 1, hi)
    base = offs[lvl][:, None]
    w = wl_i[lvl][:, None]
    rb = jnp.stack([base + y0 * w, base + y1i * w], axis=1)
    xx = jnp.stack([x0, x1i], axis=1)
    yv = jnp.stack([y0, y1i], axis=1)
    wy = jnp.stack([1.0 - ly, ly], axis=1)
    wx = jnp.stack([1.0 - lx, lx], axis=1)

    def expand(a, b, op):
        t = op(a[:, :, :, None, None], b[:, None, None, :, :])
        t = jnp.transpose(t, (0, 2, 4, 1, 3))
        t = t.reshape(n, _OUT, _SR, _OUT, _SR, 2, 2)
        t = jnp.transpose(t, (0, 1, 3, 2, 4, 5, 6))
        return t.reshape(n, _NB, _K)

    gidx_full = expand(rb, xx, lambda u, v: u + v)
    wgt = expand(wy, wx, lambda u, v: u * v) * (1.0 / (_SR * _SR))

    ymin = y0[:, 0]
    xmin = x0[:, 0]
    bh = y1i[:, _P - 1] - ymin + 1
    bw = x1i[:, _P - 1] - xmin + 1
    fast = (bh <= _BB) & (bw <= _BB)

    k = jnp.arange(_FROWS, dtype=jnp.int32)
    ycell = jnp.minimum(ymin[:, None] + k[None, :] // _BB, hi)
    xcell = jnp.minimum(xmin[:, None] + k[None, :] % _BB, hi)
    bbox_gidx = offs[lvl][:, None] + ycell * wl_i[lvl][:, None] + xcell

    zero = jnp.zeros_like(yv)
    yexp = expand(yv, zero, lambda u, v: u + v)
    xexp = expand(zero, xx, lambda u, v: u + v)
    lidx_fast = (yexp - ymin[:, None, None]) * _BB + (xexp - xmin[:, None, None])
    lidx = jnp.where(fast[:, None, None], lidx_fast, gidx_full)

    meta = jnp.zeros((n, _LANES), jnp.int32).at[:, 0].set(fast.astype(jnp.int32))
    rec = jnp.concatenate([
        meta,
        bbox_gidx,
        lidx.reshape(n, _RPB),
    ], axis=1)
    return rec, wgt.reshape(n, _RPB)


def _sc_kernel(rec_hbm, w_hbm, table_hbm, out_hbm, rec_v, w_v, rows_v, out_v,
               sem):
    wid = lax.axis_index("s") * 2 + lax.axis_index("c")
    rpw = rec_hbm.shape[0] // (_NW * _REC)

    def accum_bin(b, rbase, identity):
        lv = rec_v[pl.ds(_R_LIDX + b * _K, _K)]
        wv = w_v[pl.ds(b * _K, _K)]
        accs = [jnp.zeros((_LANES,), jnp.float32)
                for _ in range(_C // _LANES)]
        for r in range(_K):
            row = (rbase + r) if identity else (rbase + lv[r])
            wr = wv[r]
            for ch in range(_C // _LANES):
                accs[ch] = accs[ch] + wr * rows_v[
                    row, pl.ds(ch * _LANES, _LANES)]
        ob = b * _C
        for ch in range(_C // _LANES):
            out_v[pl.ds(ob + ch * _LANES, _LANES)] = accs[ch]

    def roi_body(t, _):
        roi = wid * rpw + t
        pltpu.sync_copy(rec_hbm.at[pl.ds(roi * _REC, _REC)], rec_v)
        pltpu.sync_copy(w_hbm.at[pl.ds(roi * _RPB, _RPB)], w_v)
        flag = rec_v[pl.ds(0, _LANES)][0]

        @pl.when(flag == 1)
        def _fast():
            pltpu.async_copy(
                table_hbm.at[rec_v.at[pl.ds(_R_GIDX, _FROWS)]],
                rows_v.at[pl.ds(0, _FROWS)], sem.at[0]).wait()

            def bin_body(b, _):
                accum_bin(b, 0, identity=False)
                return 0

            lax.fori_loop(0, _NB, bin_body, 0)

        @pl.when(flag != 1)
        def _slow():
            def gather_start(cidx, slot):
                return pltpu.async_copy(
                    table_hbm.at[rec_v.at[pl.ds(_R_LIDX + cidx * _GR, _GR)]],
                    rows_v.at[pl.ds(slot * _GR, _GR)], sem.at[slot])

            gather_start(0, 0)

            def chunk_body(cidx, _):
                slot = lax.rem(cidx, 2)

                @pl.when(cidx + 1 < _NCHUNK)
                def _():
                    gather_start(cidx + 1, 1 - slot)

                pltpu.make_async_copy(
                    table_hbm.at[rec_v.at[pl.ds(_R_LIDX + cidx * _GR, _GR)]],
                    rows_v.at[pl.ds(slot * _GR, _GR)], sem.at[slot]).wait()

                def bin_body(bb, _):
                    accum_bin(cidx * _BC + bb, slot * _GR + bb * _K,
                              identity=True)
                    return 0

                lax.fori_loop(0, _BC, bin_body, 0)
                return 0

            lax.fori_loop(0, _NCHUNK, chunk_body, 0)

        pltpu.sync_copy(out_v, out_hbm.at[pl.ds(roi * _ROI_F, _ROI_F)])
        return 0

    lax.fori_loop(0, rpw, roi_body, 0)


@jax.jit
def _roi_align(input_0, input_1, input_2, input_3, rois, level):
    n = rois.shape[0]
    n_pad = ((n + _NW - 1) // _NW) * _NW
    table = _build_table((input_0, input_1, input_2, input_3))
    rois_p = jnp.pad(rois, ((0, n_pad - n), (0, 0)))
    level_p = jnp.pad(level, (0, n_pad - n))
    rec, wgt = _records(rois_p, level_p)
    rec = rec.reshape(-1)
    wgt = wgt.reshape(-1)

    mesh = plsc.VectorSubcoreMesh(core_axis_name="c", subcore_axis_name="s")
    run = pl.kernel(
        _sc_kernel,
        out_type=jax.ShapeDtypeStruct((n_pad * _ROI_F,), jnp.float32),
        mesh=mesh,
        scratch_types=[
            pltpu.VMEM((_REC,), jnp.int32),
            pltpu.VMEM((_RPB,), jnp.float32),
            pltpu.VMEM((2 * _GR, _C), jnp.float32),
            pltpu.VMEM((_ROI_F,), jnp.float32),
            pltpu.SemaphoreType.DMA((2,)),
        ],
    )
    out = run(rec, wgt, table)
    out = out.reshape(n_pad, _OUT, _OUT, _C)[:n]
    return jnp.transpose(out, (0, 3, 1, 2))


def kernel(input_0, input_1, input_2, input_3, rois, rois_counts, level):
    del rois_counts
    return _roi_align(input_0, input_1, input_2, input_3, rois, level)
